# Initial kernel scaffold; baseline (speedup 1.0000x reference)
import jax
import jax.numpy as jnp
from jax.experimental import pallas as pl


def kernel(x, table):
    raise NotImplementedError("write your pallas kernel here")


# speedup vs baseline: 3.8035x; 3.8035x over previous
import functools

import jax
import jax.numpy as jnp
from jax import lax
from jax.experimental import pallas as pl
from jax.experimental.pallas import tpu as pltpu
from jax.experimental.pallas import tpu_sc as plsc

_NC, _NS = 2, 16
_NW = _NC * _NS
_W = 640


def kernel(x, table):
    B, T = x.shape
    V, D = table.shape
    n = B * T
    idx = x.reshape(n).astype(jnp.int32)
    big = jnp.pad(table, ((0, 0), (0, 128 - D)))

    n_per = n // _NW
    n_chunks = n_per // _W

    mesh = plsc.VectorSubcoreMesh(core_axis_name="c", subcore_axis_name="s")

    @functools.partial(
        pl.kernel,
        out_type=jax.ShapeDtypeStruct((n, 128), jnp.float32),
        mesh=mesh,
        scratch_types=[
            pltpu.VMEM((_W,), jnp.int32),
            pltpu.VMEM((_W, 128), jnp.float32),
            pltpu.SemaphoreType.DMA,
        ],
    )
    def _gather(tab_hbm, idx_hbm, out_hbm, idx_v, buf, sem):
        wid = lax.axis_index("s") * _NC + lax.axis_index("c")
        base = wid * n_per

        @pl.loop(0, n_chunks)
        def _(c):
            off = base + c * _W
            pltpu.sync_copy(idx_hbm.at[pl.ds(off, _W)], idx_v)
            pltpu.async_copy(tab_hbm.at[idx_v], buf, sem).wait()
            pltpu.sync_copy(buf, out_hbm.at[pl.ds(off, _W)])

    out_pad = _gather(big, idx)
    return out_pad[:, :D].reshape(B, T, D)
